# baseline (speedup 1.0000x reference)
import functools

import jax
import jax.numpy as jnp
from jax import lax
from jax.experimental import pallas as pl
from jax.experimental.pallas import tpu as pltpu
from jax.experimental.pallas import tpu_sc as plsc

N = 10000
E = 320000
H = 64
T = 2

NC = 2
NS = 16
NW = NC * NS
CH = 64
EPT = 10240
EPAD = EPT * NW
NCHUNK = EPT // CH
NPAD = 10240
RPT = NPAD // NS
RB = 1000

@functools.cache
def _make_sc_scatter(w):
    nbuf = 5
    dist = 2
    groups = NCHUNK // nbuf
    ypt = N // NS

    @functools.partial(
        pl.kernel,
        out_type=jax.ShapeDtypeStruct((NC, NPAD, w), jnp.float32),
        mesh=plsc.VectorSubcoreMesh(core_axis_name="c", subcore_axis_name="s"),
        scratch_types=[
            pltpu.VMEM((NCHUNK, CH), jnp.int32),
            pltpu.VMEM((NCHUNK, CH), jnp.int32),
            pltpu.VMEM((nbuf, CH, w), jnp.float32),
            pltpu.VMEM_SHARED((NPAD, w), jnp.float32),
            pltpu.VMEM_SHARED((NPAD, w), jnp.float32),
            [pltpu.SemaphoreType.DMA] * nbuf,
            [pltpu.SemaphoreType.DMA] * nbuf,
        ],
        compiler_params=pltpu.CompilerParams(use_tc_tiling_on_sc=False),
        name=f"sc_scatter_{w}",
    )
    def body(src_hbm, dst_hbm, zeros_hbm, y_hbm, out_hbm, sidx, didx, rows,
             acc, ytab, gsems, ssems):
        c = lax.axis_index("c")
        s = lax.axis_index("s")
        wid = c * NS + s
        pltpu.sync_copy(zeros_hbm, acc.at[pl.ds(s * RPT, RPT)])
        pltpu.sync_copy(y_hbm.at[pl.ds(s * ypt, ypt)],
                        ytab.at[pl.ds(s * ypt, ypt)])
        pltpu.sync_copy(src_hbm.at[wid], sidx)
        pltpu.sync_copy(dst_hbm.at[wid], didx)
        plsc.subcore_barrier()

        def start_g(g, b):
            pltpu.async_copy(ytab.at[sidx.at[g]], rows.at[b], gsems[b])

        def wait_g(b):
            pltpu.make_async_copy(ytab.at[sidx.at[0]], rows.at[b],
                                  gsems[b]).wait()

        def start_s(g, b):
            pltpu.async_copy(rows.at[b], acc.at[didx.at[g]], ssems[b],
                             add=True)

        def wait_s(b):
            pltpu.make_async_copy(rows.at[b], acc.at[didx.at[0]],
                                  ssems[b]).wait()

        for b in range(dist):
            start_g(b, b)

        def group(h, carry):
            for b in range(nbuf):
                g = h * nbuf + b
                wait_g(b)
                start_s(g, b)
                gn = g + dist
                bn = (b + dist) % nbuf

                @pl.when(gn < nbuf)
                def _():
                    start_g(gn, bn)

                @pl.when((gn >= nbuf) & (gn < NCHUNK))
                def _():
                    wait_s(bn)
                    start_g(gn, bn)
            return carry

        lax.fori_loop(0, groups, group, 0)
        for b in range(nbuf):
            wait_s(b)
        plsc.subcore_barrier()
        pltpu.sync_copy(acc.at[pl.ds(s * RPT, RPT)],
                        out_hbm.at[c, pl.ds(s * RPT, RPT)])

    return body


@functools.cache
def _make_sc_degree():
    @functools.partial(
        pl.kernel,
        out_type=jax.ShapeDtypeStruct((NC, NPAD, 16), jnp.float32),
        mesh=plsc.VectorSubcoreMesh(core_axis_name="c", subcore_axis_name="s"),
        scratch_types=[
            pltpu.VMEM((NCHUNK, CH), jnp.int32),
            pltpu.VMEM((CH, 16), jnp.float32),
            pltpu.VMEM_SHARED((NPAD, 16), jnp.float32),
            [pltpu.SemaphoreType.DMA] * 4,
        ],
        compiler_params=pltpu.CompilerParams(use_tc_tiling_on_sc=False),
        name="sc_degree",
    )
    def body(dst_hbm, ones_hbm, zeros_hbm, out_hbm, didx, ones_v, acc, ssems):
        c = lax.axis_index("c")
        s = lax.axis_index("s")
        wid = c * NS + s
        pltpu.sync_copy(ones_hbm, ones_v)
        pltpu.sync_copy(zeros_hbm, acc.at[pl.ds(s * RPT, RPT)])
        pltpu.sync_copy(dst_hbm.at[wid], didx)
        plsc.subcore_barrier()

        def chunk(g, carry):
            for b in range(4):
                gg = g * 4 + b

                @pl.when(gg >= 4)
                def _():
                    pltpu.make_async_copy(ones_v, acc.at[didx.at[0]],
                                          ssems[b]).wait()

                pltpu.async_copy(ones_v, acc.at[didx.at[gg]], ssems[b],
                                 add=True)
            return carry

        lax.fori_loop(0, NCHUNK // 4, chunk, 0)
        for b in range(4):
            pltpu.make_async_copy(ones_v, acc.at[didx.at[0]], ssems[b]).wait()
        plsc.subcore_barrier()
        pltpu.sync_copy(acc.at[pl.ds(s * RPT, RPT)],
                        out_hbm.at[c, pl.ds(s * RPT, RPT)])

    return body


def _row_spec(width):
    return pl.BlockSpec((RB, width), lambda i: (i, 0))


def _full_spec(shape):
    return pl.BlockSpec(shape, lambda i: tuple(0 for _ in shape))


def _stage0_body(d0_ref, d1_ref, x0_ref, dinv_ref, v_ref):
    deg = d0_ref[:, 0:1] + d1_ref[:, 0:1] + 1.0
    dinv = lax.rsqrt(deg)
    dinv_ref[...] = dinv
    v_ref[...] = dinv * x0_ref[...]


_stage0 = pl.pallas_call(
    _stage0_body,
    grid=(N // RB,),
    in_specs=[_row_spec(16), _row_spec(16), _row_spec(H)],
    out_specs=[_row_spec(1), _row_spec(H)],
    out_shape=[
        jax.ShapeDtypeStruct((N, 1), jnp.float32),
        jax.ShapeDtypeStruct((N, H), jnp.float32),
    ],
)


def _stage_a_body(s0_ref, s1_ref, v_ref, dinv_ref, w_ref, b_ref,
                  vhs_ref, vht_ref):
    dinv = dinv_ref[...]
    agg = dinv * (s0_ref[...] + s1_ref[...] + v_ref[...])
    hs = jnp.maximum(
        jnp.dot(agg, w_ref[0], preferred_element_type=jnp.float32)
        + b_ref[0], 0.0)
    ht = jnp.maximum(
        jnp.dot(agg, w_ref[1], preferred_element_type=jnp.float32)
        + b_ref[1], 0.0)
    vhs_ref[...] = dinv * hs
    vht_ref[...] = dinv * ht


_stage_a = pl.pallas_call(
    _stage_a_body,
    grid=(N // RB,),
    in_specs=[
        _row_spec(H), _row_spec(H), _row_spec(H), _row_spec(1),
        _full_spec((2, H, H)), _full_spec((2, 1, H)),
    ],
    out_specs=[_row_spec(H), _row_spec(H)],
    out_shape=[jax.ShapeDtypeStruct((N, H), jnp.float32),
               jax.ShapeDtypeStruct((N, H), jnp.float32)],
)


def _stage_b_body(s0s_ref, s1s_ref, s0t_ref, s1t_ref, vhs_ref, vht_ref,
                  dinv_ref, w_ref, b_ref, xo_ref, xn_ref, vn_ref, ldj_ref):
    dinv = dinv_ref[...]
    aggs = dinv * (s0s_ref[...] + s1s_ref[...] + vhs_ref[...])
    aggt = dinv * (s0t_ref[...] + s1t_ref[...] + vht_ref[...])
    s_ = jax.nn.sigmoid(
        jnp.dot(aggs, w_ref[0], preferred_element_type=jnp.float32)
        + b_ref[0])
    t_ = jax.nn.sigmoid(
        jnp.dot(aggt, w_ref[1], preferred_element_type=jnp.float32)
        + b_ref[1])
    xn = xo_ref[...] * jnp.exp(s_) + t_
    xn_ref[...] = xn
    vn_ref[...] = dinv * xn

    @pl.when(pl.program_id(0) == 0)
    def _():
        ldj_ref[...] = jnp.zeros((1, 1), jnp.float32)

    ldj_ref[...] += jnp.sum(s_).reshape(1, 1)


_stage_b = pl.pallas_call(
    _stage_b_body,
    grid=(N // RB,),
    in_specs=[
        _row_spec(H), _row_spec(H), _row_spec(H), _row_spec(H),
        _row_spec(H), _row_spec(H), _row_spec(1),
        _full_spec((2, H, H)), _full_spec((2, 1, H)), _row_spec(H),
    ],
    out_specs=[_row_spec(H), _row_spec(H),
               pl.BlockSpec((1, 1), lambda i: (0, 0))],
    out_shape=[
        jax.ShapeDtypeStruct((N, H), jnp.float32),
        jax.ShapeDtypeStruct((N, H), jnp.float32),
        jax.ShapeDtypeStruct((1, 1), jnp.float32),
    ],
)


def kernel(x, edge_index, Ws, bs):
    ei = edge_index.astype(jnp.int32)
    pad = EPAD - E
    src = jnp.concatenate([ei[0], jnp.zeros((pad,), jnp.int32)])
    src = src.reshape(NW, NCHUNK, CH)
    dst = jnp.concatenate([ei[1], jnp.full((pad,), N, jnp.int32)])
    dst = dst.reshape(NW, NCHUNK, CH)

    ones16 = jnp.ones((CH, 16), jnp.float32)
    zeros16 = jnp.zeros((RPT, 16), jnp.float32)
    zeros64 = jnp.zeros((RPT, H), jnp.float32)

    degp = _make_sc_degree()(dst, ones16, zeros16)
    x0, x1 = x[:, :H], x[:, H:]
    dinv, v = _stage0(degp[0, :N], degp[1, :N], x0)

    cur = [x0, x1]
    ldj_parts = []
    for half in range(2 * T):
        it = half // 2
        if half % 2 == 0:
            si, ti, upd = 0 * T + it, 1 * T + it, 1
        else:
            si, ti, upd = 2 * T + it, 3 * T + it, 0
        w1 = jnp.stack([Ws[si, 0], Ws[ti, 0]])
        b1 = jnp.stack([bs[si, 0], bs[ti, 0]])[:, None, :]
        w2 = jnp.stack([Ws[si, 1], Ws[ti, 1]])
        b2 = jnp.stack([bs[si, 1], bs[ti, 1]])[:, None, :]

        sp = _make_sc_scatter(H)(src, dst, zeros64, v)
        vhs, vht = _stage_a(sp[0, :N], sp[1, :N], v, dinv, w1, b1)
        sps = _make_sc_scatter(H)(src, dst, zeros64, vhs)
        spt = _make_sc_scatter(H)(src, dst, zeros64, vht)
        xn, vn, ldjp = _stage_b(sps[0, :N], sps[1, :N], spt[0, :N],
                                spt[1, :N], vhs, vht, dinv, w2, b2,
                                cur[upd])
        cur[upd] = xn
        v = vn
        ldj_parts.append(ldjp[0, 0])

    out = jnp.concatenate(cur, axis=1)
    ldj = ldj_parts[0] + ldj_parts[1] + ldj_parts[2] + ldj_parts[3]
    return out, ldj


# speedup vs baseline: 1.0881x; 1.0881x over previous
import functools

import jax
import jax.numpy as jnp
from jax import lax
from jax.experimental import pallas as pl
from jax.experimental.pallas import tpu as pltpu
from jax.experimental.pallas import tpu_sc as plsc

N = 10000
E = 320000
H = 64
T = 2

NC = 2
NS = 16
NW = NC * NS
CH = 80
EPT = 10240
EPAD = EPT * NW
NCHUNK = EPT // CH
NPAD = 10240
RPT = NPAD // NS
RB = 1000

@functools.cache
def _make_sc_scatter(w):
    nbuf = 4
    dist = 2
    groups = NCHUNK // nbuf
    ypt = N // NS

    @functools.partial(
        pl.kernel,
        out_type=jax.ShapeDtypeStruct((NC, NPAD, w), jnp.float32),
        mesh=plsc.VectorSubcoreMesh(core_axis_name="c", subcore_axis_name="s"),
        scratch_types=[
            pltpu.VMEM((NCHUNK, CH), jnp.int32),
            pltpu.VMEM((NCHUNK, CH), jnp.int32),
            pltpu.VMEM((nbuf, CH, w), jnp.float32),
            pltpu.VMEM_SHARED((NPAD, w), jnp.float32),
            pltpu.VMEM_SHARED((NPAD, w), jnp.float32),
            [pltpu.SemaphoreType.DMA] * nbuf,
            [pltpu.SemaphoreType.DMA] * nbuf,
        ],
        compiler_params=pltpu.CompilerParams(use_tc_tiling_on_sc=False),
        name=f"sc_scatter_{w}",
    )
    def body(src_hbm, dst_hbm, zeros_hbm, y_hbm, out_hbm, sidx, didx, rows,
             acc, ytab, gsems, ssems):
        c = lax.axis_index("c")
        s = lax.axis_index("s")
        wid = c * NS + s
        pltpu.sync_copy(zeros_hbm, acc.at[pl.ds(s * RPT, RPT)])
        pltpu.sync_copy(y_hbm.at[pl.ds(s * ypt, ypt)],
                        ytab.at[pl.ds(s * ypt, ypt)])
        pltpu.sync_copy(src_hbm.at[wid], sidx)
        pltpu.sync_copy(dst_hbm.at[wid], didx)
        plsc.subcore_barrier()

        def start_g(g, b):
            pltpu.async_copy(ytab.at[sidx.at[g]], rows.at[b], gsems[b])

        def wait_g(b):
            pltpu.make_async_copy(ytab.at[sidx.at[0]], rows.at[b],
                                  gsems[b]).wait()

        def start_s(g, b):
            pltpu.async_copy(rows.at[b], acc.at[didx.at[g]], ssems[b],
                             add=True)

        def wait_s(b):
            pltpu.make_async_copy(rows.at[b], acc.at[didx.at[0]],
                                  ssems[b]).wait()

        for b in range(dist):
            start_g(b, b)

        def group(h, carry):
            for b in range(nbuf):
                g = h * nbuf + b
                wait_g(b)
                start_s(g, b)
                gn = g + dist
                bn = (b + dist) % nbuf

                @pl.when(gn < nbuf)
                def _():
                    start_g(gn, bn)

                @pl.when((gn >= nbuf) & (gn < NCHUNK))
                def _():
                    wait_s(bn)
                    start_g(gn, bn)
            return carry

        lax.fori_loop(0, groups, group, 0)
        for b in range(nbuf):
            wait_s(b)
        plsc.subcore_barrier()
        pltpu.sync_copy(acc.at[pl.ds(s * RPT, RPT)],
                        out_hbm.at[c, pl.ds(s * RPT, RPT)])

    return body


@functools.cache
def _make_sc_degree():
    @functools.partial(
        pl.kernel,
        out_type=jax.ShapeDtypeStruct((NC, NPAD, 16), jnp.float32),
        mesh=plsc.VectorSubcoreMesh(core_axis_name="c", subcore_axis_name="s"),
        scratch_types=[
            pltpu.VMEM((NCHUNK, CH), jnp.int32),
            pltpu.VMEM((CH, 16), jnp.float32),
            pltpu.VMEM_SHARED((NPAD, 16), jnp.float32),
            [pltpu.SemaphoreType.DMA] * 4,
        ],
        compiler_params=pltpu.CompilerParams(use_tc_tiling_on_sc=False),
        name="sc_degree",
    )
    def body(dst_hbm, ones_hbm, zeros_hbm, out_hbm, didx, ones_v, acc, ssems):
        c = lax.axis_index("c")
        s = lax.axis_index("s")
        wid = c * NS + s
        pltpu.sync_copy(ones_hbm, ones_v)
        pltpu.sync_copy(zeros_hbm, acc.at[pl.ds(s * RPT, RPT)])
        pltpu.sync_copy(dst_hbm.at[wid], didx)
        plsc.subcore_barrier()

        def chunk(g, carry):
            for b in range(4):
                gg = g * 4 + b

                @pl.when(gg >= 4)
                def _():
                    pltpu.make_async_copy(ones_v, acc.at[didx.at[0]],
                                          ssems[b]).wait()

                pltpu.async_copy(ones_v, acc.at[didx.at[gg]], ssems[b],
                                 add=True)
            return carry

        lax.fori_loop(0, NCHUNK // 4, chunk, 0)
        for b in range(4):
            pltpu.make_async_copy(ones_v, acc.at[didx.at[0]], ssems[b]).wait()
        plsc.subcore_barrier()
        pltpu.sync_copy(acc.at[pl.ds(s * RPT, RPT)],
                        out_hbm.at[c, pl.ds(s * RPT, RPT)])

    return body


def _row_spec(width):
    return pl.BlockSpec((RB, width), lambda i: (i, 0))


def _part_spec(c, width):
    return pl.BlockSpec((1, RB, width), lambda i, c=c: (c, i, 0))


def _full_spec(shape):
    return pl.BlockSpec(shape, lambda i: tuple(0 for _ in shape))


def _stage0_body(d0_ref, d1_ref, x0_ref, dinv_ref, v_ref):
    deg = d0_ref[0, :, 0:1] + d1_ref[0, :, 0:1] + 1.0
    dinv = lax.rsqrt(deg)
    dinv_ref[...] = dinv
    v_ref[...] = dinv * x0_ref[...]


_stage0 = pl.pallas_call(
    _stage0_body,
    grid=(N // RB,),
    in_specs=[_part_spec(0, 16), _part_spec(1, 16), _row_spec(H)],
    out_specs=[_row_spec(1), _row_spec(H)],
    out_shape=[
        jax.ShapeDtypeStruct((N, 1), jnp.float32),
        jax.ShapeDtypeStruct((N, H), jnp.float32),
    ],
)


def _stage_a_body(s0_ref, s1_ref, v_ref, dinv_ref, w_ref, b_ref,
                  vhs_ref, vht_ref):
    dinv = dinv_ref[...]
    agg = dinv * (s0_ref[0] + s1_ref[0] + v_ref[...])
    hs = jnp.maximum(
        jnp.dot(agg, w_ref[0], preferred_element_type=jnp.float32)
        + b_ref[0], 0.0)
    ht = jnp.maximum(
        jnp.dot(agg, w_ref[1], preferred_element_type=jnp.float32)
        + b_ref[1], 0.0)
    vhs_ref[...] = dinv * hs
    vht_ref[...] = dinv * ht


_stage_a = pl.pallas_call(
    _stage_a_body,
    grid=(N // RB,),
    in_specs=[
        _part_spec(0, H), _part_spec(1, H), _row_spec(H), _row_spec(1),
        _full_spec((2, H, H)), _full_spec((2, 1, H)),
    ],
    out_specs=[_row_spec(H), _row_spec(H)],
    out_shape=[jax.ShapeDtypeStruct((N, H), jnp.float32),
               jax.ShapeDtypeStruct((N, H), jnp.float32)],
)


def _stage_b_body(s0s_ref, s1s_ref, s0t_ref, s1t_ref, vhs_ref, vht_ref,
                  dinv_ref, w_ref, b_ref, xo_ref, xn_ref, vn_ref, ldj_ref):
    dinv = dinv_ref[...]
    aggs = dinv * (s0s_ref[0] + s1s_ref[0] + vhs_ref[...])
    aggt = dinv * (s0t_ref[0] + s1t_ref[0] + vht_ref[...])
    s_ = jax.nn.sigmoid(
        jnp.dot(aggs, w_ref[0], preferred_element_type=jnp.float32)
        + b_ref[0])
    t_ = jax.nn.sigmoid(
        jnp.dot(aggt, w_ref[1], preferred_element_type=jnp.float32)
        + b_ref[1])
    xn = xo_ref[...] * jnp.exp(s_) + t_
    xn_ref[...] = xn
    vn_ref[...] = dinv * xn

    @pl.when(pl.program_id(0) == 0)
    def _():
        ldj_ref[...] = jnp.zeros((1, 1), jnp.float32)

    ldj_ref[...] += jnp.sum(s_).reshape(1, 1)


_stage_b = pl.pallas_call(
    _stage_b_body,
    grid=(N // RB,),
    in_specs=[
        _part_spec(0, H), _part_spec(1, H), _part_spec(0, H), _part_spec(1, H),
        _row_spec(H), _row_spec(H), _row_spec(1),
        _full_spec((2, H, H)), _full_spec((2, 1, H)), _row_spec(H),
    ],
    out_specs=[_row_spec(H), _row_spec(H),
               pl.BlockSpec((1, 1), lambda i: (0, 0))],
    out_shape=[
        jax.ShapeDtypeStruct((N, H), jnp.float32),
        jax.ShapeDtypeStruct((N, H), jnp.float32),
        jax.ShapeDtypeStruct((1, 1), jnp.float32),
    ],
)


def kernel(x, edge_index, Ws, bs):
    ei = edge_index.astype(jnp.int32)
    pad = EPAD - E
    src = jnp.concatenate([ei[0], jnp.zeros((pad,), jnp.int32)])
    src = src.reshape(NW, NCHUNK, CH)
    dst = jnp.concatenate([ei[1], jnp.full((pad,), N, jnp.int32)])
    dst = dst.reshape(NW, NCHUNK, CH)

    ones16 = jnp.ones((CH, 16), jnp.float32)
    zeros16 = jnp.zeros((RPT, 16), jnp.float32)
    zeros64 = jnp.zeros((RPT, H), jnp.float32)

    degp = _make_sc_degree()(dst, ones16, zeros16)
    x0, x1 = x[:, :H], x[:, H:]
    dinv, v = _stage0(degp, degp, x0)

    cur = [x0, x1]
    ldj_parts = []
    for half in range(2 * T):
        it = half // 2
        if half % 2 == 0:
            si, ti, upd = 0 * T + it, 1 * T + it, 1
        else:
            si, ti, upd = 2 * T + it, 3 * T + it, 0
        w1 = jnp.stack([Ws[si, 0], Ws[ti, 0]])
        b1 = jnp.stack([bs[si, 0], bs[ti, 0]])[:, None, :]
        w2 = jnp.stack([Ws[si, 1], Ws[ti, 1]])
        b2 = jnp.stack([bs[si, 1], bs[ti, 1]])[:, None, :]

        sp = _make_sc_scatter(H)(src, dst, zeros64, v)
        vhs, vht = _stage_a(sp, sp, v, dinv, w1, b1)
        sps = _make_sc_scatter(H)(src, dst, zeros64, vhs)
        spt = _make_sc_scatter(H)(src, dst, zeros64, vht)
        xn, vn, ldjp = _stage_b(sps, sps, spt, spt, vhs, vht, dinv, w2, b2,
                                cur[upd])
        cur[upd] = xn
        v = vn
        ldj_parts.append(ldjp[0, 0])

    out = jnp.concatenate(cur, axis=1)
    ldj = ldj_parts[0] + ldj_parts[1] + ldj_parts[2] + ldj_parts[3]
    return out, ldj


# speedup vs baseline: 1.1041x; 1.0147x over previous
import functools

import jax
import jax.numpy as jnp
from jax import lax
from jax.experimental import pallas as pl
from jax.experimental.pallas import tpu as pltpu
from jax.experimental.pallas import tpu_sc as plsc

N = 10000
E = 320000
H = 64
T = 2

NC = 2
NS = 16
NW = NC * NS
CH = 80
EPT = 10240
EPAD = EPT * NW
NCHUNK = EPT // CH
NPAD = 10240
RPT = NPAD // NS
RB = 2000

@functools.cache
def _make_sc_scatter(w):
    nbuf = 4
    dist = 2
    groups = NCHUNK // nbuf
    ypt = N // NS

    @functools.partial(
        pl.kernel,
        out_type=jax.ShapeDtypeStruct((NC, NPAD, w), jnp.float32),
        mesh=plsc.VectorSubcoreMesh(core_axis_name="c", subcore_axis_name="s"),
        scratch_types=[
            pltpu.VMEM((NCHUNK, CH), jnp.int32),
            pltpu.VMEM((NCHUNK, CH), jnp.int32),
            pltpu.VMEM((nbuf, CH, w), jnp.float32),
            pltpu.VMEM_SHARED((NPAD, w), jnp.float32),
            pltpu.VMEM_SHARED((NPAD, w), jnp.float32),
            [pltpu.SemaphoreType.DMA] * nbuf,
            [pltpu.SemaphoreType.DMA] * nbuf,
        ],
        compiler_params=pltpu.CompilerParams(use_tc_tiling_on_sc=False),
        name=f"sc_scatter_{w}",
    )
    def body(src_hbm, dst_hbm, zeros_hbm, y_hbm, out_hbm, sidx, didx, rows,
             acc, ytab, gsems, ssems):
        c = lax.axis_index("c")
        s = lax.axis_index("s")
        wid = c * NS + s
        pltpu.sync_copy(zeros_hbm, acc.at[pl.ds(s * RPT, RPT)])
        pltpu.sync_copy(y_hbm.at[pl.ds(s * ypt, ypt)],
                        ytab.at[pl.ds(s * ypt, ypt)])
        pltpu.sync_copy(src_hbm.at[wid], sidx)
        pltpu.sync_copy(dst_hbm.at[wid], didx)
        plsc.subcore_barrier()

        def start_g(g, b):
            pltpu.async_copy(ytab.at[sidx.at[g]], rows.at[b], gsems[b])

        def wait_g(b):
            pltpu.make_async_copy(ytab.at[sidx.at[0]], rows.at[b],
                                  gsems[b]).wait()

        def start_s(g, b):
            pltpu.async_copy(rows.at[b], acc.at[didx.at[g]], ssems[b],
                             add=True)

        def wait_s(b):
            pltpu.make_async_copy(rows.at[b], acc.at[didx.at[0]],
                                  ssems[b]).wait()

        for b in range(dist):
            start_g(b, b)

        def group(h, carry):
            for b in range(nbuf):
                g = h * nbuf + b
                wait_g(b)
                start_s(g, b)
                gn = g + dist
                bn = (b + dist) % nbuf

                @pl.when(gn < nbuf)
                def _():
                    start_g(gn, bn)

                @pl.when((gn >= nbuf) & (gn < NCHUNK))
                def _():
                    wait_s(bn)
                    start_g(gn, bn)
            return carry

        lax.fori_loop(0, groups, group, 0)
        for b in range(nbuf):
            wait_s(b)
        plsc.subcore_barrier()
        pltpu.sync_copy(acc.at[pl.ds(s * RPT, RPT)],
                        out_hbm.at[c, pl.ds(s * RPT, RPT)])

    return body


@functools.cache
def _make_sc_degree():
    @functools.partial(
        pl.kernel,
        out_type=jax.ShapeDtypeStruct((NC, NPAD, 16), jnp.float32),
        mesh=plsc.VectorSubcoreMesh(core_axis_name="c", subcore_axis_name="s"),
        scratch_types=[
            pltpu.VMEM((NCHUNK, CH), jnp.int32),
            pltpu.VMEM((CH, 16), jnp.float32),
            pltpu.VMEM_SHARED((NPAD, 16), jnp.float32),
            [pltpu.SemaphoreType.DMA] * 4,
        ],
        compiler_params=pltpu.CompilerParams(use_tc_tiling_on_sc=False),
        name="sc_degree",
    )
    def body(dst_hbm, ones_hbm, zeros_hbm, out_hbm, didx, ones_v, acc, ssems):
        c = lax.axis_index("c")
        s = lax.axis_index("s")
        wid = c * NS + s
        pltpu.sync_copy(ones_hbm, ones_v)
        pltpu.sync_copy(zeros_hbm, acc.at[pl.ds(s * RPT, RPT)])
        pltpu.sync_copy(dst_hbm.at[wid], didx)
        plsc.subcore_barrier()

        def chunk(g, carry):
            for b in range(4):
                gg = g * 4 + b

                @pl.when(gg >= 4)
                def _():
                    pltpu.make_async_copy(ones_v, acc.at[didx.at[0]],
                                          ssems[b]).wait()

                pltpu.async_copy(ones_v, acc.at[didx.at[gg]], ssems[b],
                                 add=True)
            return carry

        lax.fori_loop(0, NCHUNK // 4, chunk, 0)
        for b in range(4):
            pltpu.make_async_copy(ones_v, acc.at[didx.at[0]], ssems[b]).wait()
        plsc.subcore_barrier()
        pltpu.sync_copy(acc.at[pl.ds(s * RPT, RPT)],
                        out_hbm.at[c, pl.ds(s * RPT, RPT)])

    return body


def _row_spec(width):
    return pl.BlockSpec((RB, width), lambda i: (i, 0))


def _part_spec(c, width):
    return pl.BlockSpec((1, RB, width), lambda i, c=c: (c, i, 0))


def _full_spec(shape):
    return pl.BlockSpec(shape, lambda i: tuple(0 for _ in shape))


def _stage0_body(d0_ref, d1_ref, x0_ref, dinv_ref, v_ref):
    deg = d0_ref[0, :, 0:1] + d1_ref[0, :, 0:1] + 1.0
    dinv = lax.rsqrt(deg)
    dinv_ref[...] = dinv
    v_ref[...] = dinv * x0_ref[...]


_stage0 = pl.pallas_call(
    _stage0_body,
    grid=(N // RB,),
    in_specs=[_part_spec(0, 16), _part_spec(1, 16), _row_spec(H)],
    out_specs=[_row_spec(1), _row_spec(H)],
    out_shape=[
        jax.ShapeDtypeStruct((N, 1), jnp.float32),
        jax.ShapeDtypeStruct((N, H), jnp.float32),
    ],
)


def _stage_a_body(s0_ref, s1_ref, v_ref, dinv_ref, w_ref, b_ref,
                  vhs_ref, vht_ref):
    dinv = dinv_ref[...]
    agg = dinv * (s0_ref[0] + s1_ref[0] + v_ref[...])
    hs = jnp.maximum(
        jnp.dot(agg, w_ref[0], preferred_element_type=jnp.float32)
        + b_ref[0], 0.0)
    ht = jnp.maximum(
        jnp.dot(agg, w_ref[1], preferred_element_type=jnp.float32)
        + b_ref[1], 0.0)
    vhs_ref[...] = dinv * hs
    vht_ref[...] = dinv * ht


_stage_a = pl.pallas_call(
    _stage_a_body,
    grid=(N // RB,),
    in_specs=[
        _part_spec(0, H), _part_spec(1, H), _row_spec(H), _row_spec(1),
        _full_spec((2, H, H)), _full_spec((2, 1, H)),
    ],
    out_specs=[_row_spec(H), _row_spec(H)],
    out_shape=[jax.ShapeDtypeStruct((N, H), jnp.float32),
               jax.ShapeDtypeStruct((N, H), jnp.float32)],
)


def _stage_b_body(s0s_ref, s1s_ref, s0t_ref, s1t_ref, vhs_ref, vht_ref,
                  dinv_ref, w_ref, b_ref, xo_ref, xn_ref, vn_ref, ldj_ref):
    dinv = dinv_ref[...]
    aggs = dinv * (s0s_ref[0] + s1s_ref[0] + vhs_ref[...])
    aggt = dinv * (s0t_ref[0] + s1t_ref[0] + vht_ref[...])
    s_ = jax.nn.sigmoid(
        jnp.dot(aggs, w_ref[0], preferred_element_type=jnp.float32)
        + b_ref[0])
    t_ = jax.nn.sigmoid(
        jnp.dot(aggt, w_ref[1], preferred_element_type=jnp.float32)
        + b_ref[1])
    xn = xo_ref[...] * jnp.exp(s_) + t_
    xn_ref[...] = xn
    vn_ref[...] = dinv * xn

    @pl.when(pl.program_id(0) == 0)
    def _():
        ldj_ref[...] = jnp.zeros((1, 1), jnp.float32)

    ldj_ref[...] += jnp.sum(s_).reshape(1, 1)


_stage_b = pl.pallas_call(
    _stage_b_body,
    grid=(N // RB,),
    in_specs=[
        _part_spec(0, H), _part_spec(1, H), _part_spec(0, H), _part_spec(1, H),
        _row_spec(H), _row_spec(H), _row_spec(1),
        _full_spec((2, H, H)), _full_spec((2, 1, H)), _row_spec(H),
    ],
    out_specs=[_row_spec(H), _row_spec(H),
               pl.BlockSpec((1, 1), lambda i: (0, 0))],
    out_shape=[
        jax.ShapeDtypeStruct((N, H), jnp.float32),
        jax.ShapeDtypeStruct((N, H), jnp.float32),
        jax.ShapeDtypeStruct((1, 1), jnp.float32),
    ],
)


def kernel(x, edge_index, Ws, bs):
    ei = edge_index.astype(jnp.int32)
    pad = EPAD - E
    src = jnp.concatenate([ei[0], jnp.zeros((pad,), jnp.int32)])
    src = src.reshape(NW, NCHUNK, CH)
    dst = jnp.concatenate([ei[1], jnp.full((pad,), N, jnp.int32)])
    dst = dst.reshape(NW, NCHUNK, CH)

    ones16 = jnp.ones((CH, 16), jnp.float32)
    zeros16 = jnp.zeros((RPT, 16), jnp.float32)
    zeros64 = jnp.zeros((RPT, H), jnp.float32)

    degp = _make_sc_degree()(dst, ones16, zeros16)
    x0, x1 = x[:, :H], x[:, H:]
    dinv, v = _stage0(degp, degp, x0)

    cur = [x0, x1]
    ldj_parts = []
    for half in range(2 * T):
        it = half // 2
        if half % 2 == 0:
            si, ti, upd = 0 * T + it, 1 * T + it, 1
        else:
            si, ti, upd = 2 * T + it, 3 * T + it, 0
        w1 = jnp.stack([Ws[si, 0], Ws[ti, 0]])
        b1 = jnp.stack([bs[si, 0], bs[ti, 0]])[:, None, :]
        w2 = jnp.stack([Ws[si, 1], Ws[ti, 1]])
        b2 = jnp.stack([bs[si, 1], bs[ti, 1]])[:, None, :]

        sp = _make_sc_scatter(H)(src, dst, zeros64, v)
        vhs, vht = _stage_a(sp, sp, v, dinv, w1, b1)
        sps = _make_sc_scatter(H)(src, dst, zeros64, vhs)
        spt = _make_sc_scatter(H)(src, dst, zeros64, vht)
        xn, vn, ldjp = _stage_b(sps, sps, spt, spt, vhs, vht, dinv, w2, b2,
                                cur[upd])
        cur[upd] = xn
        v = vn
        ldj_parts.append(ldjp[0, 0])

    out = jnp.concatenate(cur, axis=1)
    ldj = ldj_parts[0] + ldj_parts[1] + ldj_parts[2] + ldj_parts[3]
    return out, ldj


# speedup vs baseline: 1.1263x; 1.0201x over previous
import functools

import jax
import jax.numpy as jnp
from jax import lax
from jax.experimental import pallas as pl
from jax.experimental.pallas import tpu as pltpu
from jax.experimental.pallas import tpu_sc as plsc

N = 10000
E = 320000
H = 64
T = 2

NC = 2
NS = 16
NW = NC * NS
CH = 80
EPT = 10240
EPAD = EPT * NW
NCHUNK = EPT // CH
NPAD = 10240
RPT = NPAD // NS
RB = 2000

@functools.cache
def _make_sc_scatter(w):
    nbuf = 4
    dist = 2
    groups = NCHUNK // nbuf
    ypt = N // NS

    @functools.partial(
        pl.kernel,
        out_type=jax.ShapeDtypeStruct((NC, NPAD, w), jnp.float32),
        mesh=plsc.VectorSubcoreMesh(core_axis_name="c", subcore_axis_name="s"),
        scratch_types=[
            pltpu.VMEM((NCHUNK, CH), jnp.int32),
            pltpu.VMEM((NCHUNK, CH), jnp.int32),
            pltpu.VMEM((nbuf, CH, w), jnp.float32),
            pltpu.VMEM_SHARED((NPAD, w), jnp.float32),
            pltpu.VMEM_SHARED((NPAD, w), jnp.float32),
            [pltpu.SemaphoreType.DMA] * nbuf,
            [pltpu.SemaphoreType.DMA] * nbuf,
        ],
        compiler_params=pltpu.CompilerParams(use_tc_tiling_on_sc=False),
        name=f"sc_scatter_{w}",
    )
    def body(src_hbm, dst_hbm, zeros_hbm, y_hbm, out_hbm, sidx, didx, rows,
             acc, ytab, gsems, ssems):
        c = lax.axis_index("c")
        s = lax.axis_index("s")
        wid = c * NS + s
        pltpu.async_copy(zeros_hbm, acc.at[pl.ds(s * RPT, RPT)], gsems[0])
        pltpu.async_copy(y_hbm.at[pl.ds(s * ypt, ypt)],
                         ytab.at[pl.ds(s * ypt, ypt)], gsems[1])
        pltpu.async_copy(src_hbm.at[wid], sidx, gsems[2])
        pltpu.async_copy(dst_hbm.at[wid], didx, gsems[3])
        pltpu.make_async_copy(zeros_hbm, acc.at[pl.ds(s * RPT, RPT)],
                              gsems[0]).wait()
        pltpu.make_async_copy(y_hbm.at[pl.ds(s * ypt, ypt)],
                              ytab.at[pl.ds(s * ypt, ypt)], gsems[1]).wait()
        pltpu.make_async_copy(src_hbm.at[wid], sidx, gsems[2]).wait()
        pltpu.make_async_copy(dst_hbm.at[wid], didx, gsems[3]).wait()
        plsc.subcore_barrier()

        def start_g(g, b):
            pltpu.async_copy(ytab.at[sidx.at[g]], rows.at[b], gsems[b])

        def wait_g(b):
            pltpu.make_async_copy(ytab.at[sidx.at[0]], rows.at[b],
                                  gsems[b]).wait()

        def start_s(g, b):
            pltpu.async_copy(rows.at[b], acc.at[didx.at[g]], ssems[b],
                             add=True)

        def wait_s(b):
            pltpu.make_async_copy(rows.at[b], acc.at[didx.at[0]],
                                  ssems[b]).wait()

        for b in range(dist):
            start_g(b, b)

        def group(h, carry):
            for b in range(nbuf):
                g = h * nbuf + b
                wait_g(b)
                start_s(g, b)
                gn = g + dist
                bn = (b + dist) % nbuf

                @pl.when(gn < nbuf)
                def _():
                    start_g(gn, bn)

                @pl.when((gn >= nbuf) & (gn < NCHUNK))
                def _():
                    wait_s(bn)
                    start_g(gn, bn)
            return carry

        lax.fori_loop(0, groups, group, 0)
        for b in range(nbuf):
            wait_s(b)
        plsc.subcore_barrier()
        pltpu.sync_copy(acc.at[pl.ds(s * RPT, RPT)],
                        out_hbm.at[c, pl.ds(s * RPT, RPT)])

    return body


@functools.cache
def _make_sc_degree():
    @functools.partial(
        pl.kernel,
        out_type=jax.ShapeDtypeStruct((NC, NPAD, 16), jnp.float32),
        mesh=plsc.VectorSubcoreMesh(core_axis_name="c", subcore_axis_name="s"),
        scratch_types=[
            pltpu.VMEM((NCHUNK, CH), jnp.int32),
            pltpu.VMEM((CH, 16), jnp.float32),
            pltpu.VMEM_SHARED((NPAD, 16), jnp.float32),
            [pltpu.SemaphoreType.DMA] * 4,
        ],
        compiler_params=pltpu.CompilerParams(use_tc_tiling_on_sc=False),
        name="sc_degree",
    )
    def body(dst_hbm, ones_hbm, zeros_hbm, out_hbm, didx, ones_v, acc, ssems):
        c = lax.axis_index("c")
        s = lax.axis_index("s")
        wid = c * NS + s
        pltpu.sync_copy(ones_hbm, ones_v)
        pltpu.sync_copy(zeros_hbm, acc.at[pl.ds(s * RPT, RPT)])
        pltpu.sync_copy(dst_hbm.at[wid], didx)
        plsc.subcore_barrier()

        def chunk(g, carry):
            for b in range(4):
                gg = g * 4 + b

                @pl.when(gg >= 4)
                def _():
                    pltpu.make_async_copy(ones_v, acc.at[didx.at[0]],
                                          ssems[b]).wait()

                pltpu.async_copy(ones_v, acc.at[didx.at[gg]], ssems[b],
                                 add=True)
            return carry

        lax.fori_loop(0, NCHUNK // 4, chunk, 0)
        for b in range(4):
            pltpu.make_async_copy(ones_v, acc.at[didx.at[0]], ssems[b]).wait()
        plsc.subcore_barrier()
        pltpu.sync_copy(acc.at[pl.ds(s * RPT, RPT)],
                        out_hbm.at[c, pl.ds(s * RPT, RPT)])

    return body


def _row_spec(width):
    return pl.BlockSpec((RB, width), lambda i: (i, 0))


def _part_spec(c, width):
    return pl.BlockSpec((1, RB, width), lambda i, c=c: (c, i, 0))


def _full_spec(shape):
    return pl.BlockSpec(shape, lambda i: tuple(0 for _ in shape))


def _stage0_body(d0_ref, d1_ref, x0_ref, dinv_ref, v_ref):
    deg = d0_ref[0, :, 0:1] + d1_ref[0, :, 0:1] + 1.0
    dinv = lax.rsqrt(deg)
    dinv_ref[...] = dinv
    v_ref[...] = dinv * x0_ref[...]


_stage0 = pl.pallas_call(
    _stage0_body,
    grid=(N // RB,),
    in_specs=[_part_spec(0, 16), _part_spec(1, 16), _row_spec(H)],
    out_specs=[_row_spec(1), _row_spec(H)],
    out_shape=[
        jax.ShapeDtypeStruct((N, 1), jnp.float32),
        jax.ShapeDtypeStruct((N, H), jnp.float32),
    ],
)


def _stage_a_body(s0_ref, s1_ref, v_ref, dinv_ref, w_ref, b_ref,
                  vhs_ref, vht_ref):
    dinv = dinv_ref[...]
    agg = dinv * (s0_ref[0] + s1_ref[0] + v_ref[...])
    hs = jnp.maximum(
        jnp.dot(agg, w_ref[0], preferred_element_type=jnp.float32)
        + b_ref[0], 0.0)
    ht = jnp.maximum(
        jnp.dot(agg, w_ref[1], preferred_element_type=jnp.float32)
        + b_ref[1], 0.0)
    vhs_ref[...] = dinv * hs
    vht_ref[...] = dinv * ht


_stage_a = pl.pallas_call(
    _stage_a_body,
    grid=(N // RB,),
    in_specs=[
        _part_spec(0, H), _part_spec(1, H), _row_spec(H), _row_spec(1),
        _full_spec((2, H, H)), _full_spec((2, 1, H)),
    ],
    out_specs=[_row_spec(H), _row_spec(H)],
    out_shape=[jax.ShapeDtypeStruct((N, H), jnp.float32),
               jax.ShapeDtypeStruct((N, H), jnp.float32)],
)


def _stage_b_body(s0s_ref, s1s_ref, s0t_ref, s1t_ref, vhs_ref, vht_ref,
                  dinv_ref, w_ref, b_ref, xo_ref, xn_ref, vn_ref, ldj_ref):
    dinv = dinv_ref[...]
    aggs = dinv * (s0s_ref[0] + s1s_ref[0] + vhs_ref[...])
    aggt = dinv * (s0t_ref[0] + s1t_ref[0] + vht_ref[...])
    s_ = jax.nn.sigmoid(
        jnp.dot(aggs, w_ref[0], preferred_element_type=jnp.float32)
        + b_ref[0])
    t_ = jax.nn.sigmoid(
        jnp.dot(aggt, w_ref[1], preferred_element_type=jnp.float32)
        + b_ref[1])
    xn = xo_ref[...] * jnp.exp(s_) + t_
    xn_ref[...] = xn
    vn_ref[...] = dinv * xn

    @pl.when(pl.program_id(0) == 0)
    def _():
        ldj_ref[...] = jnp.zeros((1, 1), jnp.float32)

    ldj_ref[...] += jnp.sum(s_).reshape(1, 1)


_stage_b = pl.pallas_call(
    _stage_b_body,
    grid=(N // RB,),
    in_specs=[
        _part_spec(0, H), _part_spec(1, H), _part_spec(0, H), _part_spec(1, H),
        _row_spec(H), _row_spec(H), _row_spec(1),
        _full_spec((2, H, H)), _full_spec((2, 1, H)), _row_spec(H),
    ],
    out_specs=[_row_spec(H), _row_spec(H),
               pl.BlockSpec((1, 1), lambda i: (0, 0))],
    out_shape=[
        jax.ShapeDtypeStruct((N, H), jnp.float32),
        jax.ShapeDtypeStruct((N, H), jnp.float32),
        jax.ShapeDtypeStruct((1, 1), jnp.float32),
    ],
)


def kernel(x, edge_index, Ws, bs):
    ei = edge_index.astype(jnp.int32)
    pad = EPAD - E
    src = jnp.concatenate([ei[0], jnp.zeros((pad,), jnp.int32)])
    src = src.reshape(NW, NCHUNK, CH)
    dst = jnp.concatenate([ei[1], jnp.full((pad,), N, jnp.int32)])
    dst = dst.reshape(NW, NCHUNK, CH)

    ones16 = jnp.ones((CH, 16), jnp.float32)
    zeros16 = jnp.zeros((RPT, 16), jnp.float32)
    zeros64 = jnp.zeros((RPT, H), jnp.float32)

    degp = _make_sc_degree()(dst, ones16, zeros16)
    x0, x1 = x[:, :H], x[:, H:]
    dinv, v = _stage0(degp, degp, x0)

    cur = [x0, x1]
    ldj_parts = []
    for half in range(2 * T):
        it = half // 2
        if half % 2 == 0:
            si, ti, upd = 0 * T + it, 1 * T + it, 1
        else:
            si, ti, upd = 2 * T + it, 3 * T + it, 0
        w1 = jnp.stack([Ws[si, 0], Ws[ti, 0]])
        b1 = jnp.stack([bs[si, 0], bs[ti, 0]])[:, None, :]
        w2 = jnp.stack([Ws[si, 1], Ws[ti, 1]])
        b2 = jnp.stack([bs[si, 1], bs[ti, 1]])[:, None, :]

        sp = _make_sc_scatter(H)(src, dst, zeros64, v)
        vhs, vht = _stage_a(sp, sp, v, dinv, w1, b1)
        sps = _make_sc_scatter(H)(src, dst, zeros64, vhs)
        spt = _make_sc_scatter(H)(src, dst, zeros64, vht)
        xn, vn, ldjp = _stage_b(sps, sps, spt, spt, vhs, vht, dinv, w2, b2,
                                cur[upd])
        cur[upd] = xn
        v = vn
        ldj_parts.append(ldjp[0, 0])

    out = jnp.concatenate(cur, axis=1)
    ldj = ldj_parts[0] + ldj_parts[1] + ldj_parts[2] + ldj_parts[3]
    return out, ldj
